# baseline (speedup 1.0000x reference)
import functools

import numpy as np
import jax
import jax.numpy as jnp
from jax import lax
from jax.experimental import pallas as pl
from jax.experimental.pallas import tpu as pltpu
from jax.experimental.pallas import tpu_sc as plsc

_MAX_LEN = 200
_HIDDEN = 64
_LANES = 16


def _pos_encoding_np():
    pos = np.arange(_MAX_LEN, dtype=np.float32).reshape(-1, 1)
    div = np.power(
        10000.0, np.arange(0, _HIDDEN, 2, dtype=np.float32) / _HIDDEN
    )
    ang = pos / div
    P = np.zeros((_MAX_LEN, _HIDDEN), dtype=np.float32)
    P[:, 0::2] = np.sin(ang)
    P[:, 1::2] = np.cos(ang)
    return P


@jax.jit
def _encoder_sc(idx, table, pe):
    B, two, half = idx.shape
    L = two * half
    V, H = table.shape
    NW = 32
    bpw = B // NW
    mesh = plsc.VectorSubcoreMesh(core_axis_name="c", subcore_axis_name="s")

    @functools.partial(
        pl.kernel,
        mesh=mesh,
        compiler_params=pltpu.CompilerParams(use_tc_tiling_on_sc=False),
        out_type=jax.ShapeDtypeStruct((B, L, H), jnp.float32),
        scratch_types=[
            pltpu.VMEM((2, half), jnp.int32),
            pltpu.VMEM((L, H), jnp.float32),
            pltpu.VMEM((_MAX_LEN, _HIDDEN), jnp.float32),
            pltpu.SemaphoreType.DMA,
        ],
    )
    def k(idx_hbm, table_hbm, pe_hbm, out_hbm, idx_v, rows_v, pe_v, sem):
        wid = lax.axis_index("s") * 2 + lax.axis_index("c")
        pltpu.sync_copy(pe_hbm, pe_v)

        @pl.loop(0, bpw)
        def _(i):
            b = wid * bpw + i
            pltpu.sync_copy(idx_hbm.at[b], idx_v)
            cp0 = pltpu.async_copy(
                table_hbm.at[idx_v.at[0]], rows_v.at[pl.ds(0, half)], sem
            )
            cp1 = pltpu.async_copy(
                table_hbm.at[idx_v.at[1]], rows_v.at[pl.ds(half, half)], sem
            )
            cp0.wait()
            cp1.wait()

            @pl.loop(0, L)
            def _(r):
                @pl.loop(0, H, step=_LANES)
                def _(h):
                    rows_v[r, pl.ds(h, _LANES)] += pe_v[r, pl.ds(h, _LANES)]

            pltpu.sync_copy(rows_v, out_hbm.at[b])

    return k(idx, table, pe)


def kernel(x, table):
    B, L = x.shape
    idx = x.astype(jnp.int32).reshape(B, 2, L // 2)
    pe = jnp.asarray(_pos_encoding_np())
    return _encoder_sc(idx, table, pe)


# speedup vs baseline: 1.2054x; 1.2054x over previous
import functools

import numpy as np
import jax
import jax.numpy as jnp
from jax import lax
from jax.experimental import pallas as pl
from jax.experimental.pallas import tpu as pltpu
from jax.experimental.pallas import tpu_sc as plsc

_MAX_LEN = 200
_HIDDEN = 64
_LANES = 16
_NBUF = 4


def _pos_encoding_np():
    pos = np.arange(_MAX_LEN, dtype=np.float32).reshape(-1, 1)
    div = np.power(
        10000.0, np.arange(0, _HIDDEN, 2, dtype=np.float32) / _HIDDEN
    )
    ang = pos / div
    P = np.zeros((_MAX_LEN, _HIDDEN), dtype=np.float32)
    P[:, 0::2] = np.sin(ang)
    P[:, 1::2] = np.cos(ang)
    return P


@jax.jit
def _encoder_sc(idx, table, pe):
    NW, bpw, two, half = idx.shape
    L = two * half
    B = NW * bpw
    V, H = table.shape
    mesh = plsc.VectorSubcoreMesh(core_axis_name="c", subcore_axis_name="s")

    @functools.partial(
        pl.kernel,
        mesh=mesh,
        compiler_params=pltpu.CompilerParams(use_tc_tiling_on_sc=False),
        out_type=jax.ShapeDtypeStruct((B, L, H), jnp.float32),
        scratch_types=[
            pltpu.VMEM((bpw, two, half), jnp.int32),
            pltpu.VMEM((_MAX_LEN, _HIDDEN), jnp.float32),
        ]
        + [pltpu.VMEM((L, H), jnp.float32) for _ in range(_NBUF)]
        + [pltpu.SemaphoreType.DMA for _ in range(2 * _NBUF)],
    )
    def k(idx_hbm, table_hbm, pe_hbm, out_hbm, idx_v, pe_v, r0, r1, r2, r3,
          g0, g1, g2, g3, w0, w1, w2, w3):
        rows = [r0, r1, r2, r3]
        sem_g = [g0, g1, g2, g3]
        sem_w = [w0, w1, w2, w3]
        wid = lax.axis_index("s") * 2 + lax.axis_index("c")
        pltpu.sync_copy(pe_hbm, pe_v)
        pltpu.sync_copy(idx_hbm.at[wid], idx_v)

        def issue_gather(i, b):
            pltpu.async_copy(
                table_hbm.at[idx_v.at[i, 0]],
                rows[b].at[pl.ds(0, half)], sem_g[b]
            )
            pltpu.async_copy(
                table_hbm.at[idx_v.at[i, 1]],
                rows[b].at[pl.ds(half, half)], sem_g[b]
            )

        def drain(sem, b):
            pltpu.make_async_copy(
                table_hbm.at[pl.ds(0, L)], rows[b], sem
            ).wait()

        def add_pe(b):
            @pl.loop(0, L, step=4)
            def _(r):
                for rr in range(4):
                    for h in range(0, H, _LANES):
                        rows[b][r + rr, pl.ds(h, _LANES)] += (
                            pe_v[r + rr, pl.ds(h, _LANES)]
                        )

        def issue_write(i, b):
            pltpu.async_copy(rows[b], out_hbm.at[wid * bpw + i], sem_w[b])

        def step(i, b, do_issue, do_drain_w):
            b2 = (b + 2) % _NBUF
            if do_drain_w:
                drain(sem_w[b2], b2)
            if do_issue:
                issue_gather(i + 2, b2)
            drain(sem_g[b], b)
            add_pe(b)
            issue_write(i, b)

        issue_gather(0, 0)
        issue_gather(1, 1)
        step(0, 0, True, False)
        step(1, 1, True, False)
        step(2, 2, True, True)
        step(3, 3, True, True)

        @pl.loop(1, bpw // _NBUF - 1)
        def _(g):
            i = g * _NBUF
            for b in range(_NBUF):
                step(i + b, b, True, True)

        i_last = bpw - _NBUF
        step(i_last + 0, 0, True, True)
        step(i_last + 1, 1, True, True)
        step(i_last + 2, 2, False, False)
        step(i_last + 3, 3, False, False)
        for b in range(_NBUF):
            drain(sem_w[b], b)

    return k(idx, table, pe)


def kernel(x, table):
    B, L = x.shape
    idx = x.astype(jnp.int32).reshape(32, B // 32, 2, L // 2)
    pe = jnp.asarray(_pos_encoding_np())
    return _encoder_sc(idx, table, pe)
